# baseline (speedup 1.0000x reference)
import functools

import jax
import jax.numpy as jnp
from jax import lax
from jax.experimental import pallas as pl
from jax.experimental.pallas import tpu as pltpu
from jax.experimental.pallas import tpu_sc as plsc

B = 16384
NBITS = 8
D = 256
V = 256


ROWS_PER_STEP = 2048
N_STEPS = B // ROWS_PER_STEP


def _tc_index_body(x_ref, wt_ref, idx_ref):
    x = x_ref[...]
    wt = wt_ref[...]
    logits = jnp.dot(x, wt, preferred_element_type=jnp.float32)
    probs = jax.nn.softmax(logits, axis=-1)
    idx_ref[0, 0, :] = jnp.argmax(probs, axis=-1).astype(jnp.int32)


def _compute_indices(x, wt):
    idx3 = pl.pallas_call(
        _tc_index_body,
        grid=(N_STEPS,),
        in_specs=[
            pl.BlockSpec((ROWS_PER_STEP, NBITS), lambda i: (i, 0)),
            pl.BlockSpec((NBITS, D), lambda i: (0, 0)),
        ],
        out_specs=pl.BlockSpec((1, 1, ROWS_PER_STEP), lambda i: (i, 0, 0)),
        out_shape=jax.ShapeDtypeStruct((N_STEPS, 1, ROWS_PER_STEP), jnp.int32),
    )(x, wt)
    return idx3


_NC = 2
_NS = 16
_NW = _NC * _NS
_BPW = B // _NW
_CH = 64
_NCHUNK = _BPW // _CH


def _sc_gather_body(table_hbm, idx_hbm, out_hbm, table_v, idx_v,
                    rows_v0, rows_v1, tsem, wsem0, wsem1):
    wid = lax.axis_index("s") * _NC + lax.axis_index("c")
    base = wid * _BPW
    _PER_STEP_W = ROWS_PER_STEP // _BPW

    tload = pltpu.async_copy(table_hbm, table_v, tsem)
    pltpu.sync_copy(
        idx_hbm.at[wid // _PER_STEP_W, 0,
                   pl.ds((wid % _PER_STEP_W) * _BPW, _BPW)], idx_v)
    tload.wait()

    bufs = (rows_v0, rows_v1)
    wsems = (wsem0, wsem1)
    writes = [None, None]
    for c in range(_NCHUNK):
        b = c % 2
        if writes[b] is not None:
            writes[b].wait()
        out_v = bufs[b]

        def _group(g, carry):
            row16 = idx_v[pl.ds(c * _CH + g * 16, 16)]
            dstbase = g * 16
            for l in range(16):
                src = row16[l]
                dst = dstbase + l
                for k in range(D // 16):
                    out_v[dst, pl.ds(k * 16, 16)] = (
                        table_v[src, pl.ds(k * 16, 16)])
            return carry

        lax.fori_loop(0, _CH // 16, _group, 0)
        writes[b] = pltpu.async_copy(
            out_v, out_hbm.at[pl.ds(base + c * _CH, _CH)], wsems[b])
    for w in writes:
        if w is not None:
            w.wait()


@functools.lru_cache(maxsize=1)
def _make_sc_gather():
    mesh = plsc.VectorSubcoreMesh(
        core_axis_name="c", subcore_axis_name="s",
        num_cores=_NC, num_subcores=_NS)
    return pl.kernel(
        _sc_gather_body,
        out_type=jax.ShapeDtypeStruct((B, D), jnp.float32),
        mesh=mesh,
        compiler_params=pltpu.CompilerParams(needs_layout_passes=False),
        scratch_types=[
            pltpu.VMEM((V, D), jnp.float32),
            pltpu.VMEM((_BPW,), jnp.int32),
            pltpu.VMEM((_CH, D), jnp.float32),
            pltpu.VMEM((_CH, D), jnp.float32),
            pltpu.SemaphoreType.DMA,
            pltpu.SemaphoreType.DMA,
            pltpu.SemaphoreType.DMA,
        ],
    )


def kernel(binary_signature, W_b2i, odu_table):
    bs = binary_signature
    idx3 = _compute_indices(bs, W_b2i.T)
    wisdom = _make_sc_gather()(odu_table, idx3)
    return (bs, idx3.reshape(B), wisdom)
